# baseline (speedup 1.0000x reference)
import functools

import jax
import jax.numpy as jnp
from jax import lax
from jax.experimental import pallas as pl
from jax.experimental.pallas import tpu as pltpu
from jax.experimental.pallas import tpu_sc as plsc

HC = 128
N = 50000
PADN = 50176
RPT = 3136
E = 500000
EC = 1000
EP = 507904
ECS = 512
NCC = 992
ECA = 256
NDBL = 62
DUMP = 50160
FSB = 112
EL = 200000
NCHL = 200
KMAXL = 7
NEG = 0.01
BLK = 512
GRID = PADN // BLK

_mesh = functools.partial(
    plsc.VectorSubcoreMesh, core_axis_name="c", subcore_axis_name="s")


def _lrelu(x):
    return jnp.where(x >= 0, x, NEG * x)


def _prep_body(emb_d_ref, dx_ref, w_ref, b_ref, emb_s_ref, *out_refs):
    xd = emb_d_ref[...]
    xs = (jnp.dot(dx_ref[...], w_ref[...], preferred_element_type=jnp.float32)
          + b_ref[...] + emb_s_ref[...])
    for c in range(4):
        out_refs[c][...] = xd[:, 32 * c:32 * (c + 1)]
        out_refs[4 + c][...] = xs[:, 32 * c:32 * (c + 1)]


def _prep(emb_drug, dx16, w16, b, emb_dis):
    outs = [jax.ShapeDtypeStruct((PADN, 32), jnp.float32)] * 8
    return pl.pallas_call(
        _prep_body,
        grid=(GRID,),
        in_specs=[
            pl.BlockSpec((BLK, HC), lambda i: (i, 0)),
            pl.BlockSpec((BLK, 16), lambda i: (i, 0)),
            pl.BlockSpec((16, HC), lambda i: (0, 0)),
            pl.BlockSpec((1, HC), lambda i: (0, 0)),
            pl.BlockSpec((BLK, HC), lambda i: (i, 0)),
        ],
        out_specs=[pl.BlockSpec((BLK, 32), lambda i: (i, 0))] * 8,
        out_shape=outs,
    )(emb_drug, dx16, w16, b.reshape(1, HC), emb_dis)


def _mm2_body(x0, x1, x2, x3, y0, y1, y2, y3, wl_ref, wr_ref, bl_ref, *out_refs):
    xp = jnp.concatenate([x0[...], x1[...], x2[...], x3[...]], axis=1)
    xr = jnp.concatenate([y0[...], y1[...], y2[...], y3[...]], axis=1)
    p = jnp.dot(xp, wl_ref[...], preferred_element_type=jnp.float32)
    r = jnp.dot(xr, wr_ref[...], preferred_element_type=jnp.float32) + bl_ref[...]
    for c in range(4):
        out_refs[c][...] = p[:, 32 * c:32 * (c + 1)]
        out_refs[4 + c][...] = r[:, 32 * c:32 * (c + 1)]


def _mm2(xp4, xr4, wl, wr, bl):
    outs = [jax.ShapeDtypeStruct((PADN, 32), jnp.float32)] * 8
    res = pl.pallas_call(
        _mm2_body,
        grid=(GRID,),
        in_specs=(
            [pl.BlockSpec((BLK, 32), lambda i: (i, 0))] * 8
            + [pl.BlockSpec((HC, HC), lambda i: (0, 0))] * 2
            + [pl.BlockSpec((1, HC), lambda i: (0, 0))]
        ),
        out_specs=[pl.BlockSpec((BLK, 32), lambda i: (i, 0))] * 8,
        out_shape=outs,
    )(*xp4, *xr4, wl, wr, bl.reshape(1, HC))
    return res[:4], res[4:]


def _ab_body(x0, x1, x2, x3, y0, y1, y2, y3, wt_ref, wb_ref, b0_ref, a_ref, b_ref):
    xd = jnp.concatenate([x0[...], x1[...], x2[...], x3[...]], axis=1)
    xs = jnp.concatenate([y0[...], y1[...], y2[...], y3[...]], axis=1)
    a_ref[...] = (jnp.dot(xd, wt_ref[...], preferred_element_type=jnp.float32)
                  + b0_ref[...])
    b_ref[...] = jnp.dot(xs, wb_ref[...], preferred_element_type=jnp.float32)


def _ab(xd4, xs4, w0, b0):
    outs = [jax.ShapeDtypeStruct((PADN, 16), jnp.float32)] * 2
    return pl.pallas_call(
        _ab_body,
        grid=(GRID,),
        in_specs=(
            [pl.BlockSpec((BLK, 32), lambda i: (i, 0))] * 8
            + [pl.BlockSpec((HC, 16), lambda i: (0, 0))] * 2
            + [pl.BlockSpec((1, 16), lambda i: (0, 0))]
        ),
        out_specs=[pl.BlockSpec((BLK, 16), lambda i: (i, 0))] * 2,
        out_shape=outs,
    )(*xd4, *xs4, w0[:HC], w0[HC:], b0.reshape(1, 16))


def _mlp_body(h0_ref, w1_ref, b1_ref, w2_ref, b2_ref, out_ref):
    h = _lrelu(h0_ref[...])
    h = _lrelu(jnp.dot(h, w1_ref[...], preferred_element_type=jnp.float32)
               + b1_ref[...])
    o = jnp.dot(h, w2_ref[...], preferred_element_type=jnp.float32) + b2_ref[...]
    out_ref[...] = o[:, 0]


def _mlp(h0, w1, b1, w2, b2):
    blk = 4096
    grid = (EL + blk - 1) // blk
    return pl.pallas_call(
        _mlp_body,
        grid=(grid,),
        in_specs=[
            pl.BlockSpec((blk, 16), lambda i: (i, 0)),
            pl.BlockSpec((16, 8), lambda i: (0, 0)),
            pl.BlockSpec((1, 8), lambda i: (0, 0)),
            pl.BlockSpec((8, 1), lambda i: (0, 0)),
            pl.BlockSpec((1, 1), lambda i: (0, 0)),
        ],
        out_specs=pl.BlockSpec((blk,), lambda i: (i,)),
        out_shape=jax.ShapeDtypeStruct((EL,), jnp.float32),
    )(h0, w1, b1.reshape(1, 8), w2, b2.reshape(1, 1))


def _zero_rows(buf, nrows, width):
    z = jnp.zeros((16,), jnp.float32)

    def body(i, _):
        for h in range(width // 16):
            buf[i, pl.ds(16 * h, 16)] = z
        return 0

    lax.fori_loop(0, nrows, body, 0, unroll=False)


def _count_kernel(ddst, rdst, recd, recr, acc, onesv, idxv, cntv):
    c = lax.axis_index("c")
    s = lax.axis_index("s")
    row0 = pl.multiple_of(s * RPT, 8)
    for which in range(2):
        dref = (ddst, rdst)[which]
        oref = (recd, recr)[which]

        @pl.when(c == which)
        def _():
            _zero_rows(cntv, RPT, 16)
            pltpu.sync_copy(cntv, acc.at[pl.ds(row0, RPT)])
            one = jnp.full((16,), 1.0, jnp.float32)

            def fill(i, _):
                onesv[i] = one
                return 0

            lax.fori_loop(0, ECS, fill, 0, unroll=False)
            plsc.subcore_barrier()

            def chunk(k, _):
                cid = s + 16 * k
                off = pl.multiple_of(cid * ECS, 8)
                pltpu.sync_copy(dref.at[pl.ds(off, ECS)], idxv)
                pltpu.sync_copy(onesv, acc.at[idxv], add=True)
                return 0

            lax.fori_loop(0, NCC // 16, chunk, 0, unroll=False)
            plsc.subcore_barrier()
            pltpu.sync_copy(acc.at[pl.ds(row0, RPT)], cntv)

            def recip(j, _):
                v = cntv[j]
                cntv[j] = 1.0 / jnp.maximum(v, 1.0)
                return 0

            lax.fori_loop(0, RPT, recip, 0, unroll=False)
            pltpu.sync_copy(cntv, oref.at[pl.ds(row0, RPT)])


def _sc_count(ddst, rdst):
    k = pl.kernel(
        _count_kernel,
        mesh=_mesh(),
        compiler_params=pltpu.CompilerParams(use_tc_tiling_on_sc=False),
        out_type=[jax.ShapeDtypeStruct((PADN, 16), jnp.float32)] * 2,
        scratch_types=[
            pltpu.VMEM_SHARED((PADN, 16), jnp.float32),
            pltpu.VMEM((ECS, 16), jnp.float32),
            pltpu.VMEM((ECS,), jnp.int32),
            pltpu.VMEM((RPT, 16), jnp.float32),
        ],
    )
    return k(ddst, rdst)


def _agg_kernel(p0, p1, p2, p3, src, dst, r0, r1, r2, r3, recip,
                o0, o1, o2, o3, acc,
                rows0, rows1, src0, src1, dst0, dst1, rcv,
                gs0, gs1, ss0, ss1):
    c = lax.axis_index("c")
    s = lax.axis_index("s")
    row0 = pl.multiple_of(s * RPT, 8)
    prefs = (p0, p1, p2, p3)
    orefs = (o0, o1, o2, o3)
    rrefs = (r0, r1, r2, r3)
    rows = (rows0, rows1)
    srcv = (src0, src1)
    dstv = (dst0, dst1)
    gsem = (gs0, gs1)
    ssem = (ss0, ss1)
    for q in range(4):
        pref, rref, oref = prefs[q], rrefs[q], orefs[q]

        @pl.when(c == q % 2)
        def _():
            _zero_rows(rows0, ECA, 32)
            for z in range(RPT // ECA):
                pltpu.sync_copy(rows0, acc.at[pl.ds(row0 + z * ECA, ECA)])
            pltpu.sync_copy(rows0.at[pl.ds(0, RPT % ECA)],
                            acc.at[pl.ds(row0 + (RPT // ECA) * ECA,
                                         RPT % ECA)])
            plsc.subcore_barrier()

            def dchunk(tt, _):
                ghandles = []
                for b in range(2):
                    k = 2 * tt + b
                    cid = s + 16 * k
                    off = pl.multiple_of(cid * ECA, 8)
                    pltpu.sync_copy(src.at[pl.ds(off, ECA)], srcv[b])
                    pltpu.sync_copy(dst.at[pl.ds(off, ECA)], dstv[b])
                    ghandles.append(
                        pltpu.async_copy(pref.at[srcv[b]], rows[b], gsem[b]))
                shandles = []
                for b in range(2):
                    ghandles[b].wait()
                    shandles.append(
                        pltpu.async_copy(rows[b], acc.at[dstv[b]], ssem[b],
                                         add=True))
                for b in range(2):
                    shandles[b].wait()
                return 0

            lax.fori_loop(0, NDBL, dchunk, 0, unroll=False)
            plsc.subcore_barrier()
            for h in range(RPT // FSB):
                rr = pl.multiple_of(row0 + FSB * h, 8)
                pltpu.sync_copy(acc.at[pl.ds(rr, FSB)], rows0.at[pl.ds(0, FSB)])
                pltpu.sync_copy(rref.at[pl.ds(rr, FSB)],
                                rows0.at[pl.ds(FSB, FSB)])
                pltpu.sync_copy(recip.at[pl.ds(rr, FSB)], rcv)

                def frow(j, _):
                    rc = rcv[j]
                    for half in range(2):
                        sl = pl.ds(16 * half, 16)
                        a = rows0[j, sl] * rc + rows0[FSB + j, sl]
                        rows0[FSB + j, sl] = _lrelu(a)
                    return 0

                lax.fori_loop(0, FSB, frow, 0, unroll=False)
                pltpu.sync_copy(rows0.at[pl.ds(FSB, FSB)],
                                oref.at[pl.ds(rr, FSB)])


def _sc_agg(p4, src, dst, r4, recip):
    k = pl.kernel(
        _agg_kernel,
        mesh=_mesh(),
        compiler_params=pltpu.CompilerParams(use_tc_tiling_on_sc=False),
        out_type=[jax.ShapeDtypeStruct((PADN, 32), jnp.float32)] * 4,
        scratch_types=[
            pltpu.VMEM_SHARED((PADN, 32), jnp.float32),
            pltpu.VMEM((ECA, 32), jnp.float32),
            pltpu.VMEM((ECA, 32), jnp.float32),
            pltpu.VMEM((ECA,), jnp.int32),
            pltpu.VMEM((ECA,), jnp.int32),
            pltpu.VMEM((ECA,), jnp.int32),
            pltpu.VMEM((ECA,), jnp.int32),
            pltpu.VMEM((FSB, 16), jnp.float32),
            pltpu.SemaphoreType.DMA,
            pltpu.SemaphoreType.DMA,
            pltpu.SemaphoreType.DMA,
            pltpu.SemaphoreType.DMA,
        ],
    )
    return list(k(*p4, src, dst, *r4, recip))


def _pair_kernel(a, b, i0, i1, h0, i0v, i1v, av, bv):
    c = lax.axis_index("c")
    s = lax.axis_index("s")
    wid = s * 2 + c

    def chunk(k, _):
        cid = wid + 32 * k

        @pl.when(cid < NCHL)
        def _():
            off = pl.multiple_of(cid * EC, 8)
            pltpu.sync_copy(i0.at[pl.ds(off, EC)], i0v)
            pltpu.sync_copy(i1.at[pl.ds(off, EC)], i1v)
            pltpu.sync_copy(a.at[i0v], av)
            pltpu.sync_copy(b.at[i1v], bv)

            def add(j, _):
                av[j] = av[j] + bv[j]
                return 0

            lax.fori_loop(0, EC, add, 0, unroll=False)
            pltpu.sync_copy(av, h0.at[pl.ds(off, EC)])
        return 0

    lax.fori_loop(0, KMAXL, chunk, 0, unroll=False)


def _sc_pair(a, b, i0, i1):
    k = pl.kernel(
        _pair_kernel,
        mesh=_mesh(),
        compiler_params=pltpu.CompilerParams(use_tc_tiling_on_sc=False),
        out_type=jax.ShapeDtypeStruct((EL, 16), jnp.float32),
        scratch_types=[
            pltpu.VMEM((EC,), jnp.int32),
            pltpu.VMEM((EC,), jnp.int32),
            pltpu.VMEM((EC, 16), jnp.float32),
            pltpu.VMEM((EC, 16), jnp.float32),
        ],
    )
    return k(a, b, i0, i1)


def kernel(drug_node_id, disease_node_id, moa_node_id, epc_node_id, pe_node_id, tc_node_id, hc_node_id, apc_node_id, ext_node_id, pk_node_id, disease_x, edge_index_d2dis, edge_index_dis2d, edge_label_index, emb_drug, emb_disease, emb_moa, emb_epc, emb_pe, emb_tc, emb_hc, emb_apc, emb_ext, emb_pk, W_dis_lin, b_dis_lin, Wl_d2dis_0, bl_d2dis_0, Wr_d2dis_0, Wl_dis2d_0, bl_dis2d_0, Wr_dis2d_0, Wl_d2dis_1, bl_d2dis_1, Wr_d2dis_1, Wl_dis2d_1, bl_dis2d_1, Wr_dis2d_1, W_cls0, b_cls0, W_cls1, b_cls1, W_cls2, b_cls2):
    pad = EP - E
    sd = jnp.pad(edge_index_d2dis[0], (0, pad))
    dd = jnp.pad(edge_index_d2dis[1], (0, pad), constant_values=DUMP)
    sr = jnp.pad(edge_index_dis2d[0], (0, pad))
    dr = jnp.pad(edge_index_dis2d[1], (0, pad), constant_values=DUMP)
    eli0, eli1 = edge_label_index[0], edge_label_index[1]
    dx16 = jnp.pad(disease_x, ((0, 0), (0, 6)))
    w16 = jnp.pad(W_dis_lin, ((0, 6), (0, 0)))

    outs = _prep(emb_drug, dx16, w16, b_dis_lin, emb_disease)
    xd4, xs4 = list(outs[:4]), list(outs[4:])
    recd, recr = _sc_count(dd, dr)

    wls = {("d2dis", 0): (Wl_d2dis_0, Wr_d2dis_0, bl_d2dis_0),
           ("dis2d", 0): (Wl_dis2d_0, Wr_dis2d_0, bl_dis2d_0),
           ("d2dis", 1): (Wl_d2dis_1, Wr_d2dis_1, bl_d2dis_1),
           ("dis2d", 1): (Wl_dis2d_1, Wr_dis2d_1, bl_dis2d_1)}
    for l in range(2):
        wl, wr, bl = wls[("d2dis", l)]
        pd4, rd4 = _mm2(xd4, xs4, wl, wr, bl)
        wl, wr, bl = wls[("dis2d", l)]
        pr4, rr4 = _mm2(xs4, xd4, wl, wr, bl)
        xs4 = _sc_agg(pd4, sd, dd, rd4, recd)
        xd4 = _sc_agg(pr4, sr, dr, rr4, recr)

    a, b = _ab(xd4, xs4, W_cls0, b_cls0)
    h0 = _sc_pair(a, b, eli0, eli1)
    return _mlp(h0, W_cls1, b_cls1, W_cls2, b_cls2)


# speedup vs baseline: 1.0368x; 1.0368x over previous
import functools

import jax
import jax.numpy as jnp
from jax import lax
from jax.experimental import pallas as pl
from jax.experimental.pallas import tpu as pltpu
from jax.experimental.pallas import tpu_sc as plsc

HC = 128
N = 50000
PADN = 50176
RPT = 3136
E = 500000
EC = 1000
EP = 507904
ECS = 512
NCC = 992
ECA = 256
NDBL = 62
DUMP = 50160
FSB = 112
EL = 200000
NCHL = 200
KMAXL = 7
NEG = 0.01
BLK = 512
GRID = PADN // BLK

_mesh = functools.partial(
    plsc.VectorSubcoreMesh, core_axis_name="c", subcore_axis_name="s")


def _lrelu(x):
    return jnp.where(x >= 0, x, NEG * x)


def _prep_body(emb_d_ref, dx_ref, w_ref, b_ref, emb_s_ref, *out_refs):
    xd = emb_d_ref[...]
    xs = (jnp.dot(dx_ref[...], w_ref[...], preferred_element_type=jnp.float32)
          + b_ref[...] + emb_s_ref[...])
    for c in range(4):
        out_refs[c][...] = xd[:, 32 * c:32 * (c + 1)]
        out_refs[4 + c][...] = xs[:, 32 * c:32 * (c + 1)]


def _prep(emb_drug, dx16, w16, b, emb_dis):
    outs = [jax.ShapeDtypeStruct((PADN, 32), jnp.float32)] * 8
    return pl.pallas_call(
        _prep_body,
        grid=(GRID,),
        in_specs=[
            pl.BlockSpec((BLK, HC), lambda i: (i, 0)),
            pl.BlockSpec((BLK, 16), lambda i: (i, 0)),
            pl.BlockSpec((16, HC), lambda i: (0, 0)),
            pl.BlockSpec((1, HC), lambda i: (0, 0)),
            pl.BlockSpec((BLK, HC), lambda i: (i, 0)),
        ],
        out_specs=[pl.BlockSpec((BLK, 32), lambda i: (i, 0))] * 8,
        out_shape=outs,
    )(emb_drug, dx16, w16, b.reshape(1, HC), emb_dis)


def _mm2_body(x0, x1, x2, x3, y0, y1, y2, y3, wl_ref, wr_ref, bl_ref, *out_refs):
    xp = jnp.concatenate([x0[...], x1[...], x2[...], x3[...]], axis=1)
    xr = jnp.concatenate([y0[...], y1[...], y2[...], y3[...]], axis=1)
    p = jnp.dot(xp, wl_ref[...], preferred_element_type=jnp.float32)
    r = jnp.dot(xr, wr_ref[...], preferred_element_type=jnp.float32) + bl_ref[...]
    for c in range(4):
        out_refs[c][...] = p[:, 32 * c:32 * (c + 1)]
        out_refs[4 + c][...] = r[:, 32 * c:32 * (c + 1)]


def _mm2(xp4, xr4, wl, wr, bl):
    outs = [jax.ShapeDtypeStruct((PADN, 32), jnp.float32)] * 8
    res = pl.pallas_call(
        _mm2_body,
        grid=(GRID,),
        in_specs=(
            [pl.BlockSpec((BLK, 32), lambda i: (i, 0))] * 8
            + [pl.BlockSpec((HC, HC), lambda i: (0, 0))] * 2
            + [pl.BlockSpec((1, HC), lambda i: (0, 0))]
        ),
        out_specs=[pl.BlockSpec((BLK, 32), lambda i: (i, 0))] * 8,
        out_shape=outs,
    )(*xp4, *xr4, wl, wr, bl.reshape(1, HC))
    return res[:4], res[4:]


def _ab_body(x0, x1, x2, x3, y0, y1, y2, y3, wt_ref, wb_ref, b0_ref, a_ref, b_ref):
    xd = jnp.concatenate([x0[...], x1[...], x2[...], x3[...]], axis=1)
    xs = jnp.concatenate([y0[...], y1[...], y2[...], y3[...]], axis=1)
    a_ref[...] = (jnp.dot(xd, wt_ref[...], preferred_element_type=jnp.float32)
                  + b0_ref[...])
    b_ref[...] = jnp.dot(xs, wb_ref[...], preferred_element_type=jnp.float32)


def _ab(xd4, xs4, w0, b0):
    outs = [jax.ShapeDtypeStruct((PADN, 16), jnp.float32)] * 2
    return pl.pallas_call(
        _ab_body,
        grid=(GRID,),
        in_specs=(
            [pl.BlockSpec((BLK, 32), lambda i: (i, 0))] * 8
            + [pl.BlockSpec((HC, 16), lambda i: (0, 0))] * 2
            + [pl.BlockSpec((1, 16), lambda i: (0, 0))]
        ),
        out_specs=[pl.BlockSpec((BLK, 16), lambda i: (i, 0))] * 2,
        out_shape=outs,
    )(*xd4, *xs4, w0[:HC], w0[HC:], b0.reshape(1, 16))


def _mlp_body(h0_ref, w1_ref, b1_ref, w2_ref, b2_ref, out_ref):
    h = _lrelu(h0_ref[...])
    h = _lrelu(jnp.dot(h, w1_ref[...], preferred_element_type=jnp.float32)
               + b1_ref[...])
    o = jnp.dot(h, w2_ref[...], preferred_element_type=jnp.float32) + b2_ref[...]
    out_ref[...] = o[:, 0]


def _mlp(h0, w1, b1, w2, b2):
    blk = 4096
    grid = (EL + blk - 1) // blk
    return pl.pallas_call(
        _mlp_body,
        grid=(grid,),
        in_specs=[
            pl.BlockSpec((blk, 16), lambda i: (i, 0)),
            pl.BlockSpec((16, 8), lambda i: (0, 0)),
            pl.BlockSpec((1, 8), lambda i: (0, 0)),
            pl.BlockSpec((8, 1), lambda i: (0, 0)),
            pl.BlockSpec((1, 1), lambda i: (0, 0)),
        ],
        out_specs=pl.BlockSpec((blk,), lambda i: (i,)),
        out_shape=jax.ShapeDtypeStruct((EL,), jnp.float32),
    )(h0, w1, b1.reshape(1, 8), w2, b2.reshape(1, 1))


def _zero_rows(buf, nrows, width):
    z = jnp.zeros((16,), jnp.float32)

    def body(i, _):
        for h in range(width // 16):
            buf[i, pl.ds(16 * h, 16)] = z
        return 0

    lax.fori_loop(0, nrows, body, 0, unroll=False)


def _count_kernel(ddst, rdst, recd, recr, acc, onesv, idxv, cntv):
    c = lax.axis_index("c")
    s = lax.axis_index("s")
    row0 = pl.multiple_of(s * RPT, 8)
    for which in range(2):
        dref = (ddst, rdst)[which]
        oref = (recd, recr)[which]

        @pl.when(c == which)
        def _():
            _zero_rows(cntv, RPT, 16)
            pltpu.sync_copy(cntv, acc.at[pl.ds(row0, RPT)])
            one = jnp.full((16,), 1.0, jnp.float32)

            def fill(i, _):
                onesv[i] = one
                return 0

            lax.fori_loop(0, ECS, fill, 0, unroll=False)
            plsc.subcore_barrier()

            def chunk(k, _):
                cid = s + 16 * k
                off = pl.multiple_of(cid * ECS, 8)
                pltpu.sync_copy(dref.at[pl.ds(off, ECS)], idxv)
                pltpu.sync_copy(onesv, acc.at[idxv], add=True)
                return 0

            lax.fori_loop(0, NCC // 16, chunk, 0, unroll=False)
            plsc.subcore_barrier()
            pltpu.sync_copy(acc.at[pl.ds(row0, RPT)], cntv)

            def recip(j, _):
                v = cntv[j]
                cntv[j] = 1.0 / jnp.maximum(v, 1.0)
                return 0

            lax.fori_loop(0, RPT, recip, 0, unroll=False)
            pltpu.sync_copy(cntv, oref.at[pl.ds(row0, RPT)])


def _sc_count(ddst, rdst):
    k = pl.kernel(
        _count_kernel,
        mesh=_mesh(),
        compiler_params=pltpu.CompilerParams(use_tc_tiling_on_sc=False),
        out_type=[jax.ShapeDtypeStruct((PADN, 16), jnp.float32)] * 2,
        scratch_types=[
            pltpu.VMEM_SHARED((PADN, 16), jnp.float32),
            pltpu.VMEM((ECS, 16), jnp.float32),
            pltpu.VMEM((ECS,), jnp.int32),
            pltpu.VMEM((RPT, 16), jnp.float32),
        ],
    )
    return k(ddst, rdst)


def _agg_kernel(p0, p1, p2, p3, src, dst, r0, r1, r2, r3, recip,
                o0, o1, o2, o3, acc, rows0, src0, dst0, rcv):
    c = lax.axis_index("c")
    s = lax.axis_index("s")
    row0 = pl.multiple_of(s * RPT, 8)
    prefs = (p0, p1, p2, p3)
    orefs = (o0, o1, o2, o3)
    rrefs = (r0, r1, r2, r3)
    for q in range(4):
        pref, rref, oref = prefs[q], rrefs[q], orefs[q]

        @pl.when(c == q % 2)
        def _():
            _zero_rows(rows0, ECS, 32)
            for z in range(RPT // ECS):
                pltpu.sync_copy(rows0, acc.at[pl.ds(row0 + z * ECS, ECS)])
            pltpu.sync_copy(rows0.at[pl.ds(0, RPT % ECS)],
                            acc.at[pl.ds(row0 + (RPT // ECS) * ECS,
                                         RPT % ECS)])
            plsc.subcore_barrier()

            def chunk(k, _):
                cid = s + 16 * k
                off = pl.multiple_of(cid * ECS, 8)
                pltpu.sync_copy(src.at[pl.ds(off, ECS)], src0)
                pltpu.sync_copy(dst.at[pl.ds(off, ECS)], dst0)
                pltpu.sync_copy(pref.at[src0], rows0)
                pltpu.sync_copy(rows0, acc.at[dst0], add=True)
                return 0

            lax.fori_loop(0, NCC // 16, chunk, 0, unroll=False)
            plsc.subcore_barrier()
            for h in range(RPT // FSB):
                rr = pl.multiple_of(row0 + FSB * h, 8)
                pltpu.sync_copy(acc.at[pl.ds(rr, FSB)], rows0.at[pl.ds(0, FSB)])
                pltpu.sync_copy(rref.at[pl.ds(rr, FSB)],
                                rows0.at[pl.ds(FSB, FSB)])
                pltpu.sync_copy(recip.at[pl.ds(rr, FSB)], rcv)

                def frow(j, _):
                    rc = rcv[j]
                    for half in range(2):
                        sl = pl.ds(16 * half, 16)
                        a = rows0[j, sl] * rc + rows0[FSB + j, sl]
                        rows0[FSB + j, sl] = _lrelu(a)
                    return 0

                lax.fori_loop(0, FSB, frow, 0, unroll=False)
                pltpu.sync_copy(rows0.at[pl.ds(FSB, FSB)],
                                oref.at[pl.ds(rr, FSB)])


def _sc_agg(p4, src, dst, r4, recip):
    k = pl.kernel(
        _agg_kernel,
        mesh=_mesh(),
        compiler_params=pltpu.CompilerParams(use_tc_tiling_on_sc=False),
        out_type=[jax.ShapeDtypeStruct((PADN, 32), jnp.float32)] * 4,
        scratch_types=[
            pltpu.VMEM_SHARED((PADN, 32), jnp.float32),
            pltpu.VMEM((ECS, 32), jnp.float32),
            pltpu.VMEM((ECS,), jnp.int32),
            pltpu.VMEM((ECS,), jnp.int32),
            pltpu.VMEM((FSB, 16), jnp.float32),
        ],
    )
    return list(k(*p4, src, dst, *r4, recip))


def _pair_kernel(a, b, i0, i1, h0, i0v, i1v, av, bv):
    c = lax.axis_index("c")
    s = lax.axis_index("s")
    wid = s * 2 + c

    def chunk(k, _):
        cid = wid + 32 * k

        @pl.when(cid < NCHL)
        def _():
            off = pl.multiple_of(cid * EC, 8)
            pltpu.sync_copy(i0.at[pl.ds(off, EC)], i0v)
            pltpu.sync_copy(i1.at[pl.ds(off, EC)], i1v)
            pltpu.sync_copy(a.at[i0v], av)
            pltpu.sync_copy(b.at[i1v], bv)

            def add(j, _):
                av[j] = av[j] + bv[j]
                return 0

            lax.fori_loop(0, EC, add, 0, unroll=False)
            pltpu.sync_copy(av, h0.at[pl.ds(off, EC)])
        return 0

    lax.fori_loop(0, KMAXL, chunk, 0, unroll=False)


def _sc_pair(a, b, i0, i1):
    k = pl.kernel(
        _pair_kernel,
        mesh=_mesh(),
        compiler_params=pltpu.CompilerParams(use_tc_tiling_on_sc=False),
        out_type=jax.ShapeDtypeStruct((EL, 16), jnp.float32),
        scratch_types=[
            pltpu.VMEM((EC,), jnp.int32),
            pltpu.VMEM((EC,), jnp.int32),
            pltpu.VMEM((EC, 16), jnp.float32),
            pltpu.VMEM((EC, 16), jnp.float32),
        ],
    )
    return k(a, b, i0, i1)


def kernel(drug_node_id, disease_node_id, moa_node_id, epc_node_id, pe_node_id, tc_node_id, hc_node_id, apc_node_id, ext_node_id, pk_node_id, disease_x, edge_index_d2dis, edge_index_dis2d, edge_label_index, emb_drug, emb_disease, emb_moa, emb_epc, emb_pe, emb_tc, emb_hc, emb_apc, emb_ext, emb_pk, W_dis_lin, b_dis_lin, Wl_d2dis_0, bl_d2dis_0, Wr_d2dis_0, Wl_dis2d_0, bl_dis2d_0, Wr_dis2d_0, Wl_d2dis_1, bl_d2dis_1, Wr_d2dis_1, Wl_dis2d_1, bl_dis2d_1, Wr_dis2d_1, W_cls0, b_cls0, W_cls1, b_cls1, W_cls2, b_cls2):
    pad = EP - E
    sd = jnp.pad(edge_index_d2dis[0], (0, pad))
    dump = DUMP + (jnp.arange(pad, dtype=jnp.int32) % 16)
    dd = jnp.concatenate([edge_index_d2dis[1], dump])
    sr = jnp.pad(edge_index_dis2d[0], (0, pad))
    dr = jnp.concatenate([edge_index_dis2d[1], dump])
    eli0, eli1 = edge_label_index[0], edge_label_index[1]
    dx16 = jnp.pad(disease_x, ((0, 0), (0, 6)))
    w16 = jnp.pad(W_dis_lin, ((0, 6), (0, 0)))

    outs = _prep(emb_drug, dx16, w16, b_dis_lin, emb_disease)
    xd4, xs4 = list(outs[:4]), list(outs[4:])
    recd, recr = _sc_count(dd, dr)

    wls = {("d2dis", 0): (Wl_d2dis_0, Wr_d2dis_0, bl_d2dis_0),
           ("dis2d", 0): (Wl_dis2d_0, Wr_dis2d_0, bl_dis2d_0),
           ("d2dis", 1): (Wl_d2dis_1, Wr_d2dis_1, bl_d2dis_1),
           ("dis2d", 1): (Wl_dis2d_1, Wr_dis2d_1, bl_dis2d_1)}
    for l in range(2):
        wl, wr, bl = wls[("d2dis", l)]
        pd4, rd4 = _mm2(xd4, xs4, wl, wr, bl)
        wl, wr, bl = wls[("dis2d", l)]
        pr4, rr4 = _mm2(xs4, xd4, wl, wr, bl)
        xs4 = _sc_agg(pd4, sd, dd, rd4, recd)
        xd4 = _sc_agg(pr4, sr, dr, rr4, recr)

    a, b = _ab(xd4, xs4, W_cls0, b_cls0)
    h0 = _sc_pair(a, b, eli0, eli1)
    return _mlp(h0, W_cls1, b_cls1, W_cls2, b_cls2)


# speedup vs baseline: 1.1859x; 1.1439x over previous
import functools

import jax
import jax.numpy as jnp
from jax import lax
from jax.experimental import pallas as pl
from jax.experimental.pallas import tpu as pltpu
from jax.experimental.pallas import tpu_sc as plsc

HC = 128
N = 50000
PADN = 50176
RPT = 3136
E = 500000
EC = 1000
EP = 500224
ECS = 512
NCHS = 977
KMXS = 62
DUMP = 50160
FSB = 112
EL = 200000
NCHL = 200
KMAXL = 7
NEG = 0.01
BLK = 512
GRID = PADN // BLK

_mesh = functools.partial(
    plsc.VectorSubcoreMesh, core_axis_name="c", subcore_axis_name="s")


def _lrelu(x):
    return jnp.where(x >= 0, x, NEG * x)


def _prep_body(emb_d_ref, dx_ref, w_ref, b_ref, emb_s_ref, *out_refs):
    xd = emb_d_ref[...]
    xs = (jnp.dot(dx_ref[...], w_ref[...], preferred_element_type=jnp.float32)
          + b_ref[...] + emb_s_ref[...])
    for c in range(4):
        out_refs[c][...] = xd[:, 32 * c:32 * (c + 1)]
        out_refs[4 + c][...] = xs[:, 32 * c:32 * (c + 1)]


def _prep(emb_drug, dx16, w16, b, emb_dis):
    outs = [jax.ShapeDtypeStruct((PADN, 32), jnp.float32)] * 8
    return pl.pallas_call(
        _prep_body,
        grid=(GRID,),
        in_specs=[
            pl.BlockSpec((BLK, HC), lambda i: (i, 0)),
            pl.BlockSpec((BLK, 16), lambda i: (i, 0)),
            pl.BlockSpec((16, HC), lambda i: (0, 0)),
            pl.BlockSpec((1, HC), lambda i: (0, 0)),
            pl.BlockSpec((BLK, HC), lambda i: (i, 0)),
        ],
        out_specs=[pl.BlockSpec((BLK, 32), lambda i: (i, 0))] * 8,
        out_shape=outs,
    )(emb_drug, dx16, w16, b.reshape(1, HC), emb_dis)


def _mm2_body(x0, x1, x2, x3, y0, y1, y2, y3, wl_ref, wr_ref, bl_ref, *out_refs):
    xp = jnp.concatenate([x0[...], x1[...], x2[...], x3[...]], axis=1)
    xr = jnp.concatenate([y0[...], y1[...], y2[...], y3[...]], axis=1)
    p = jnp.dot(xp, wl_ref[...], preferred_element_type=jnp.float32)
    r = jnp.dot(xr, wr_ref[...], preferred_element_type=jnp.float32) + bl_ref[...]
    for c in range(4):
        out_refs[c][...] = p[:, 32 * c:32 * (c + 1)]
        out_refs[4 + c][...] = r[:, 32 * c:32 * (c + 1)]


def _mm2(xp4, xr4, wl, wr, bl):
    outs = [jax.ShapeDtypeStruct((PADN, 32), jnp.float32)] * 8
    res = pl.pallas_call(
        _mm2_body,
        grid=(GRID,),
        in_specs=(
            [pl.BlockSpec((BLK, 32), lambda i: (i, 0))] * 8
            + [pl.BlockSpec((HC, HC), lambda i: (0, 0))] * 2
            + [pl.BlockSpec((1, HC), lambda i: (0, 0))]
        ),
        out_specs=[pl.BlockSpec((BLK, 32), lambda i: (i, 0))] * 8,
        out_shape=outs,
    )(*xp4, *xr4, wl, wr, bl.reshape(1, HC))
    return res[:4], res[4:]


def _ab_body(x0, x1, x2, x3, y0, y1, y2, y3, wt_ref, wb_ref, b0_ref, a_ref, b_ref):
    xd = jnp.concatenate([x0[...], x1[...], x2[...], x3[...]], axis=1)
    xs = jnp.concatenate([y0[...], y1[...], y2[...], y3[...]], axis=1)
    a_ref[...] = (jnp.dot(xd, wt_ref[...], preferred_element_type=jnp.float32)
                  + b0_ref[...])
    b_ref[...] = jnp.dot(xs, wb_ref[...], preferred_element_type=jnp.float32)


def _ab(xd4, xs4, w0, b0):
    outs = [jax.ShapeDtypeStruct((PADN, 16), jnp.float32)] * 2
    return pl.pallas_call(
        _ab_body,
        grid=(GRID,),
        in_specs=(
            [pl.BlockSpec((BLK, 32), lambda i: (i, 0))] * 8
            + [pl.BlockSpec((HC, 16), lambda i: (0, 0))] * 2
            + [pl.BlockSpec((1, 16), lambda i: (0, 0))]
        ),
        out_specs=[pl.BlockSpec((BLK, 16), lambda i: (i, 0))] * 2,
        out_shape=outs,
    )(*xd4, *xs4, w0[:HC], w0[HC:], b0.reshape(1, 16))


def _mlp_body(h0_ref, w1_ref, b1_ref, w2_ref, b2_ref, out_ref):
    h = _lrelu(h0_ref[...])
    h = _lrelu(jnp.dot(h, w1_ref[...], preferred_element_type=jnp.float32)
               + b1_ref[...])
    o = jnp.dot(h, w2_ref[...], preferred_element_type=jnp.float32) + b2_ref[...]
    out_ref[...] = o[:, 0]


def _mlp(h0, w1, b1, w2, b2):
    blk = 4096
    grid = (EL + blk - 1) // blk
    return pl.pallas_call(
        _mlp_body,
        grid=(grid,),
        in_specs=[
            pl.BlockSpec((blk, 16), lambda i: (i, 0)),
            pl.BlockSpec((16, 8), lambda i: (0, 0)),
            pl.BlockSpec((1, 8), lambda i: (0, 0)),
            pl.BlockSpec((8, 1), lambda i: (0, 0)),
            pl.BlockSpec((1, 1), lambda i: (0, 0)),
        ],
        out_specs=pl.BlockSpec((blk,), lambda i: (i,)),
        out_shape=jax.ShapeDtypeStruct((EL,), jnp.float32),
    )(h0, w1, b1.reshape(1, 8), w2, b2.reshape(1, 1))


def _zero_rows(buf, nrows, width):
    z = jnp.zeros((16,), jnp.float32)

    def body(i, _):
        for h in range(width // 16):
            buf[i, pl.ds(16 * h, 16)] = z
        return 0

    lax.fori_loop(0, nrows, body, 0, unroll=False)


def _count_kernel(ddst, rdst, recd, recr, acc, onesv, idxv, cntv):
    c = lax.axis_index("c")
    s = lax.axis_index("s")
    row0 = pl.multiple_of(s * RPT, 8)
    for which in range(2):
        dref = (ddst, rdst)[which]
        oref = (recd, recr)[which]

        @pl.when(c == which)
        def _():
            _zero_rows(cntv, RPT, 16)
            pltpu.sync_copy(cntv, acc.at[pl.ds(row0, RPT)])
            one = jnp.full((16,), 1.0, jnp.float32)

            def fill(i, _):
                onesv[i] = one
                return 0

            lax.fori_loop(0, ECS, fill, 0, unroll=False)
            plsc.subcore_barrier()

            def chunk(k, _):
                cid = s + 16 * k

                @pl.when(cid < NCHS)
                def _():
                    off = pl.multiple_of(cid * ECS, 8)
                    pltpu.sync_copy(dref.at[pl.ds(off, ECS)], idxv)
                    pltpu.sync_copy(onesv, acc.at[idxv], add=True)
                return 0

            lax.fori_loop(0, KMXS, chunk, 0, unroll=False)
            plsc.subcore_barrier()
            pltpu.sync_copy(acc.at[pl.ds(row0, RPT)], cntv)

            def recip(j, _):
                v = cntv[j]
                cntv[j] = 1.0 / jnp.maximum(v, 1.0)
                return 0

            lax.fori_loop(0, RPT, recip, 0, unroll=False)
            pltpu.sync_copy(cntv, oref.at[pl.ds(row0, RPT)])


def _sc_count(ddst, rdst):
    k = pl.kernel(
        _count_kernel,
        mesh=_mesh(),
        compiler_params=pltpu.CompilerParams(use_tc_tiling_on_sc=False),
        out_type=[jax.ShapeDtypeStruct((PADN, 16), jnp.float32)] * 2,
        scratch_types=[
            pltpu.VMEM_SHARED((PADN, 16), jnp.float32),
            pltpu.VMEM((ECS, 16), jnp.float32),
            pltpu.VMEM((ECS,), jnp.int32),
            pltpu.VMEM((RPT, 16), jnp.float32),
        ],
    )
    return k(ddst, rdst)


def _agg_kernel(p0, p1, p2, p3, src, dst, r0, r1, r2, r3, recip,
                o0, o1, o2, o3, acc, rows0, src0, dst0, rcv):
    c = lax.axis_index("c")
    s = lax.axis_index("s")
    row0 = pl.multiple_of(s * RPT, 8)
    prefs = (p0, p1, p2, p3)
    orefs = (o0, o1, o2, o3)
    rrefs = (r0, r1, r2, r3)
    for q in range(4):
        pref, rref, oref = prefs[q], rrefs[q], orefs[q]

        @pl.when(c == q % 2)
        def _():
            _zero_rows(rows0, ECS, 32)
            for z in range(RPT // ECS):
                pltpu.sync_copy(rows0, acc.at[pl.ds(row0 + z * ECS, ECS)])
            pltpu.sync_copy(rows0.at[pl.ds(0, RPT % ECS)],
                            acc.at[pl.ds(row0 + (RPT // ECS) * ECS,
                                         RPT % ECS)])
            plsc.subcore_barrier()

            def chunk(k, _):
                cid = s + 16 * k

                @pl.when(cid < NCHS)
                def _():
                    off = pl.multiple_of(cid * ECS, 8)
                    pltpu.sync_copy(src.at[pl.ds(off, ECS)], src0)
                    pltpu.sync_copy(dst.at[pl.ds(off, ECS)], dst0)
                    pltpu.sync_copy(pref.at[src0], rows0)
                    pltpu.sync_copy(rows0, acc.at[dst0], add=True)
                return 0

            lax.fori_loop(0, KMXS, chunk, 0, unroll=False)
            plsc.subcore_barrier()
            for h in range(RPT // FSB):
                rr = pl.multiple_of(row0 + FSB * h, 8)
                pltpu.sync_copy(acc.at[pl.ds(rr, FSB)], rows0.at[pl.ds(0, FSB)])
                pltpu.sync_copy(rref.at[pl.ds(rr, FSB)],
                                rows0.at[pl.ds(FSB, FSB)])
                pltpu.sync_copy(recip.at[pl.ds(rr, FSB)], rcv)

                def frow(j, _):
                    rc = rcv[j]
                    for half in range(2):
                        sl = pl.ds(16 * half, 16)
                        a = rows0[j, sl] * rc + rows0[FSB + j, sl]
                        rows0[FSB + j, sl] = _lrelu(a)
                    return 0

                lax.fori_loop(0, FSB, frow, 0, unroll=False)
                pltpu.sync_copy(rows0.at[pl.ds(FSB, FSB)],
                                oref.at[pl.ds(rr, FSB)])


def _sc_agg(p4, src, dst, r4, recip):
    k = pl.kernel(
        _agg_kernel,
        mesh=_mesh(),
        compiler_params=pltpu.CompilerParams(use_tc_tiling_on_sc=False),
        out_type=[jax.ShapeDtypeStruct((PADN, 32), jnp.float32)] * 4,
        scratch_types=[
            pltpu.VMEM_SHARED((PADN, 32), jnp.float32),
            pltpu.VMEM((ECS, 32), jnp.float32),
            pltpu.VMEM((ECS,), jnp.int32),
            pltpu.VMEM((ECS,), jnp.int32),
            pltpu.VMEM((FSB, 16), jnp.float32),
        ],
    )
    return list(k(*p4, src, dst, *r4, recip))


def _pair_kernel(a, b, i0, i1, h0, i0v, i1v, av, bv):
    c = lax.axis_index("c")
    s = lax.axis_index("s")
    wid = s * 2 + c

    def chunk(k, _):
        cid = wid + 32 * k

        @pl.when(cid < NCHL)
        def _():
            off = pl.multiple_of(cid * EC, 8)
            pltpu.sync_copy(i0.at[pl.ds(off, EC)], i0v)
            pltpu.sync_copy(i1.at[pl.ds(off, EC)], i1v)
            pltpu.sync_copy(a.at[i0v], av)
            pltpu.sync_copy(b.at[i1v], bv)

            def add(j, _):
                av[j] = av[j] + bv[j]
                return 0

            lax.fori_loop(0, EC, add, 0, unroll=False)
            pltpu.sync_copy(av, h0.at[pl.ds(off, EC)])
        return 0

    lax.fori_loop(0, KMAXL, chunk, 0, unroll=False)


def _sc_pair(a, b, i0, i1):
    k = pl.kernel(
        _pair_kernel,
        mesh=_mesh(),
        compiler_params=pltpu.CompilerParams(use_tc_tiling_on_sc=False),
        out_type=jax.ShapeDtypeStruct((EL, 16), jnp.float32),
        scratch_types=[
            pltpu.VMEM((EC,), jnp.int32),
            pltpu.VMEM((EC,), jnp.int32),
            pltpu.VMEM((EC, 16), jnp.float32),
            pltpu.VMEM((EC, 16), jnp.float32),
        ],
    )
    return k(a, b, i0, i1)


def kernel(drug_node_id, disease_node_id, moa_node_id, epc_node_id, pe_node_id, tc_node_id, hc_node_id, apc_node_id, ext_node_id, pk_node_id, disease_x, edge_index_d2dis, edge_index_dis2d, edge_label_index, emb_drug, emb_disease, emb_moa, emb_epc, emb_pe, emb_tc, emb_hc, emb_apc, emb_ext, emb_pk, W_dis_lin, b_dis_lin, Wl_d2dis_0, bl_d2dis_0, Wr_d2dis_0, Wl_dis2d_0, bl_dis2d_0, Wr_dis2d_0, Wl_d2dis_1, bl_d2dis_1, Wr_d2dis_1, Wl_dis2d_1, bl_dis2d_1, Wr_dis2d_1, W_cls0, b_cls0, W_cls1, b_cls1, W_cls2, b_cls2):
    pad = EP - E
    sd = jnp.pad(edge_index_d2dis[0], (0, pad))
    dd = jnp.pad(edge_index_d2dis[1], (0, pad), constant_values=DUMP)
    sr = jnp.pad(edge_index_dis2d[0], (0, pad))
    dr = jnp.pad(edge_index_dis2d[1], (0, pad), constant_values=DUMP)
    eli0, eli1 = edge_label_index[0], edge_label_index[1]
    dx16 = jnp.pad(disease_x, ((0, 0), (0, 6)))
    w16 = jnp.pad(W_dis_lin, ((0, 6), (0, 0)))

    outs = _prep(emb_drug, dx16, w16, b_dis_lin, emb_disease)
    xd4, xs4 = list(outs[:4]), list(outs[4:])
    recd, recr = _sc_count(dd, dr)

    wls = {("d2dis", 0): (Wl_d2dis_0, Wr_d2dis_0, bl_d2dis_0),
           ("dis2d", 0): (Wl_dis2d_0, Wr_dis2d_0, bl_dis2d_0),
           ("d2dis", 1): (Wl_d2dis_1, Wr_d2dis_1, bl_d2dis_1),
           ("dis2d", 1): (Wl_dis2d_1, Wr_dis2d_1, bl_dis2d_1)}
    for l in range(2):
        wl, wr, bl = wls[("d2dis", l)]
        pd4, rd4 = _mm2(xd4, xs4, wl, wr, bl)
        wl, wr, bl = wls[("dis2d", l)]
        pr4, rr4 = _mm2(xs4, xd4, wl, wr, bl)
        xs4 = _sc_agg(pd4, sd, dd, rd4, recd)
        xd4 = _sc_agg(pr4, sr, dr, rr4, recr)

    a, b = _ab(xd4, xs4, W_cls0, b_cls0)
    h0 = _sc_pair(a, b, eli0, eli1)
    return _mlp(h0, W_cls1, b_cls1, W_cls2, b_cls2)
